# Initial kernel scaffold; baseline (speedup 1.0000x reference)
import functools

import jax
import jax.numpy as jnp
from jax import lax
from jax.experimental import pallas as pl
from jax.experimental.pallas import tpu as pltpu
from jax.experimental.pallas import tpu_sc as plsc

NC, NS = 2, 16
NW = NC * NS
K = 125
G = 64


def _sc_deg_body(n, ch, col2d, out_hbm, degS, colv, onesv, zbuf):
    c = lax.axis_index("c")
    s = lax.axis_index("s")
    wid = s * NC + c
    rpt = n // NS
    nz = rpt // K

    lane = lax.iota(jnp.int32, 16)
    e1 = jnp.where(lane == 0, jnp.float32(1), jnp.float32(0))
    z16 = jnp.zeros((16,), jnp.float32)

    def fill(r, _):
        onesv[r, :] = e1
        zbuf[r, :] = z16
        return ()
    lax.fori_loop(0, K, fill, ())

    for i in range(nz):
        pltpu.sync_copy(zbuf, degS.at[pl.ds(s * rpt + i * K, K)])
    plsc.subcore_barrier()

    pltpu.sync_copy(col2d.at[pl.ds(wid * ch, ch)], colv)

    def body(j, _):
        pltpu.sync_copy(onesv, degS.at[colv.at[j]], add=True)
        return ()
    lax.fori_loop(0, ch, body, ())
    plsc.subcore_barrier()

    for i in range(nz):
        base = s * rpt + i * K
        pltpu.sync_copy(degS.at[pl.ds(base, K)], zbuf)
        pltpu.sync_copy(zbuf, out_hbm.at[c, pl.ds(base, K)])


def _sc_agg_body(n, d, ch, h2_hbm, row2d, col2d, out_hbm,
                 aggS, rowv, colv, buf0, buf1, sem0, sem1):
    c = lax.axis_index("c")
    s = lax.axis_index("s")
    wid = s * NC + c
    rpt = n // NS
    nz = rpt // K

    z16 = jnp.zeros((16,), jnp.float32)

    def zero(i, _):
        buf0[i // (d // 16), pl.ds((i % (d // 16)) * 16, 16)] = z16
        return ()
    lax.fori_loop(0, K * (d // 16), zero, ())

    for i in range(nz):
        pltpu.sync_copy(buf0, aggS.at[pl.ds(s * rpt + i * K, K)])
    plsc.subcore_barrier()

    pltpu.sync_copy(row2d.at[pl.ds(wid * ch, ch)], rowv)
    pltpu.sync_copy(col2d.at[pl.ds(wid * ch, ch)], colv)

    pltpu.async_copy(h2_hbm.at[rowv.at[0]], buf0, sem0)

    def pair(p, _):
        j0 = 2 * p
        j1 = j0 + 1
        pltpu.async_copy(h2_hbm.at[rowv.at[j1]], buf1, sem1)
        pltpu.make_async_copy(h2_hbm.at[rowv.at[j0]], buf0, sem0).wait()
        pltpu.sync_copy(buf0, aggS.at[colv.at[j0]], add=True)

        @pl.when(j0 + 2 < ch)
        def _():
            pltpu.async_copy(h2_hbm.at[rowv.at[j0 + 2]], buf0, sem0)

        pltpu.make_async_copy(h2_hbm.at[rowv.at[j1]], buf1, sem1).wait()
        pltpu.sync_copy(buf1, aggS.at[colv.at[j1]], add=True)
        return ()
    lax.fori_loop(0, ch // 2, pair, ())
    plsc.subcore_barrier()

    for i in range(nz):
        base = s * rpt + i * K
        pltpu.sync_copy(aggS.at[pl.ds(base, K)], buf0)
        pltpu.sync_copy(buf0, out_hbm.at[c, pl.ds(base, K)])


def _tc_misc_body(d0, d1, o):
    t = d0[...] + d1[...]
    deg = jnp.sum(t, axis=1, keepdims=True) + 1.0
    o[...] = lax.rsqrt(deg)


def _tc_h2_body(x, wt, dinv, o):
    h = jnp.dot(x[...], wt[...], preferred_element_type=jnp.float32)
    o[...] = h * dinv[...]


def _tc_final_body(g, p0, p1, h2, dinv, b, gam, bet, batr, fcwt, fcb, o):
    agg = (p0[...] + p1[...] + h2[...]) * dinv[...] + b[...]
    mean = jnp.mean(agg, axis=0, keepdims=True)
    var = jnp.mean((agg - mean) ** 2, axis=0, keepdims=True)
    hn = (agg - mean) * lax.rsqrt(var + 1e-5) * gam[...] + bet[...]
    hr = jnp.maximum(hn, 0.0)
    gid = lax.broadcasted_iota(jnp.int32, (g, 1), 0)
    oh = (gid == batr[...]).astype(jnp.float32)
    sums = jnp.dot(oh, hr, preferred_element_type=jnp.float32)
    cnt = jnp.sum(oh, axis=1, keepdims=True)
    pooled = sums / jnp.maximum(cnt, 1.0)
    logits = jnp.dot(pooled, fcwt[...],
                     preferred_element_type=jnp.float32) + fcb[...]
    m = jnp.max(logits, axis=1, keepdims=True)
    ls = jnp.log(jnp.sum(jnp.exp(logits - m), axis=1, keepdims=True)) + m
    o[...] = logits - ls


def kernel(x, edge_index, batch, W, b, bn_gamma, bn_beta, fc_W, fc_b):
    n, d = x.shape
    e = edge_index.shape[1]
    cdim = fc_W.shape[0]
    epw = e // NW
    ch = epw // K
    assert e == NW * ch * K and n % (NS * K) == 0 and d % 16 == 0

    mesh = plsc.VectorSubcoreMesh(core_axis_name="c", subcore_axis_name="s")

    sc_deg = pl.kernel(
        functools.partial(_sc_deg_body, n, ch),
        out_type=jax.ShapeDtypeStruct((NC, n, 16), jnp.float32),
        mesh=mesh,
        scratch_types=[
            pltpu.VMEM_SHARED((n, 16), jnp.float32),
            pltpu.VMEM((ch, K), jnp.int32),
            pltpu.VMEM((K, 16), jnp.float32),
            pltpu.VMEM((K, 16), jnp.float32),
        ],
    )

    sc_agg = pl.kernel(
        functools.partial(_sc_agg_body, n, d, ch),
        out_type=jax.ShapeDtypeStruct((NC, n, d), jnp.float32),
        mesh=mesh,
        scratch_types=[
            pltpu.VMEM_SHARED((n, d), jnp.float32),
            pltpu.VMEM((ch, K), jnp.int32),
            pltpu.VMEM((ch, K), jnp.int32),
            pltpu.VMEM((K, d), jnp.float32),
            pltpu.VMEM((K, d), jnp.float32),
            pltpu.SemaphoreType.DMA,
            pltpu.SemaphoreType.DMA,
        ],
    )

    tc_misc = pl.pallas_call(
        _tc_misc_body,
        out_shape=jax.ShapeDtypeStruct((n, 1), jnp.float32),
    )
    tc_h2 = pl.pallas_call(
        _tc_h2_body,
        out_shape=jax.ShapeDtypeStruct((n, d), jnp.float32),
    )
    tc_final = pl.pallas_call(
        functools.partial(_tc_final_body, G),
        out_shape=jax.ShapeDtypeStruct((G, cdim), jnp.float32),
    )

    row2d = edge_index[0].reshape(e // K, K)
    col2d = edge_index[1].reshape(e // K, K)

    degp = sc_deg(col2d)
    dinv = tc_misc(degp[0], degp[1])
    h2 = tc_h2(x, W.T, dinv)
    parts = sc_agg(h2, row2d, col2d)
    return tc_final(parts[0], parts[1], h2, dinv,
                    b.reshape(1, d), bn_gamma.reshape(1, d),
                    bn_beta.reshape(1, d), batch.reshape(1, n),
                    fc_W.T, fc_b.reshape(1, cdim))


# speedup vs baseline: 27.4041x; 27.4041x over previous
import functools

import jax
import jax.numpy as jnp
from jax import lax
from jax.experimental import pallas as pl
from jax.experimental.pallas import tpu as pltpu
from jax.experimental.pallas import tpu_sc as plsc

NC, NS = 2, 16
NW = NC * NS
K = 125
KC = 64
G = 64


def _sc_deg_body(ch, rpt, col2d, out_hbm, degS, colv, onesv, zbuf):
    c = lax.axis_index("c")
    s = lax.axis_index("s")
    wid = s * NC + c
    nz = rpt // KC

    lane = lax.iota(jnp.int32, 16)
    e1 = jnp.where(lane == 0, jnp.float32(1), jnp.float32(0))
    z16 = jnp.zeros((16,), jnp.float32)

    def fill(r, _):
        onesv[r, :] = e1
        return ()
    lax.fori_loop(0, K, fill, ())

    def fillz(r, _):
        zbuf[r, :] = z16
        return ()
    lax.fori_loop(0, KC, fillz, ())

    for i in range(nz):
        pltpu.sync_copy(zbuf, degS.at[pl.ds(s * rpt + i * KC, KC)])
    plsc.subcore_barrier()

    pltpu.sync_copy(col2d.at[pl.ds(wid * ch, ch)], colv)

    def body(j, _):
        pltpu.sync_copy(onesv, degS.at[colv.at[j]], add=True)
        return ()
    lax.fori_loop(0, ch, body, ())
    plsc.subcore_barrier()

    for i in range(nz):
        base = s * rpt + i * KC
        pltpu.sync_copy(degS.at[pl.ds(base, KC)], zbuf)
        pltpu.sync_copy(zbuf, out_hbm.at[c, pl.ds(base, KC)])


BLK = 8


def _sc_agg_body(d, ch, rpt, h2_hbm, row2d, col2d, out_hbm,
                 aggS, rowv, colv, buf0, sem0):
    c = lax.axis_index("c")
    s = lax.axis_index("s")
    wid = s * NC + c
    nz = rpt // KC

    z16 = jnp.zeros((16,), jnp.float32)
    nv = d // 16

    def zero(i, _):
        buf0[i // nv, pl.ds((i % nv) * 16, 16)] = z16
        return ()
    lax.fori_loop(0, KC * nv, zero, ())

    for i in range(nz):
        pltpu.sync_copy(buf0.at[pl.ds(0, KC)],
                        aggS.at[pl.ds(s * rpt + i * KC, KC)])
    plsc.subcore_barrier()

    def blk_body(blk, _):
        base = wid * ch + blk * BLK
        pltpu.sync_copy(row2d.at[pl.ds(base, BLK)], rowv)
        pltpu.sync_copy(col2d.at[pl.ds(base, BLK)], colv)
        for t in range(BLK):
            pltpu.async_copy(h2_hbm.at[rowv.at[t]], buf0, sem0).wait()
            pltpu.sync_copy(buf0, aggS.at[colv.at[t]], add=True)
        return ()
    lax.fori_loop(0, ch // BLK, blk_body, ())
    plsc.subcore_barrier()

    for i in range(nz):
        base = s * rpt + i * KC
        pltpu.sync_copy(aggS.at[pl.ds(base, KC)], buf0.at[pl.ds(0, KC)])
        pltpu.sync_copy(buf0.at[pl.ds(0, KC)], out_hbm.at[c, pl.ds(base, KC)])


def _tc_misc_body(n, d0, d1, o):
    t = d0[...][:n] + d1[...][:n]
    deg = jnp.sum(t, axis=1, keepdims=True) + 1.0
    o[...] = lax.rsqrt(deg)


def _tc_h2_body(x, wt, dinv, o):
    h = jnp.dot(x[...], wt[...], preferred_element_type=jnp.float32)
    o[...] = h * dinv[...]


def _tc_final_body(n, g, p0, p1, h2, dinv, b, gam, bet, batr, fcwt, fcb, o):
    agg = (p0[...][:n] + p1[...][:n] + h2[...]) * dinv[...] + b[...]
    mean = jnp.mean(agg, axis=0, keepdims=True)
    var = jnp.mean((agg - mean) ** 2, axis=0, keepdims=True)
    hn = (agg - mean) * lax.rsqrt(var + 1e-5) * gam[...] + bet[...]
    hr = jnp.maximum(hn, 0.0)
    gid = lax.broadcasted_iota(jnp.int32, (g, 1), 0)
    oh = (gid == batr[...]).astype(jnp.float32)
    sums = jnp.dot(oh, hr, preferred_element_type=jnp.float32)
    cnt = jnp.sum(oh, axis=1, keepdims=True)
    pooled = sums / jnp.maximum(cnt, 1.0)
    logits = jnp.dot(pooled, fcwt[...],
                     preferred_element_type=jnp.float32) + fcb[...]
    m = jnp.max(logits, axis=1, keepdims=True)
    ls = jnp.log(jnp.sum(jnp.exp(logits - m), axis=1, keepdims=True)) + m
    o[...] = logits - ls


def kernel(x, edge_index, batch, W, b, bn_gamma, bn_beta, fc_W, fc_b):
    n, d = x.shape
    e = edge_index.shape[1]
    cdim = fc_W.shape[0]
    epw = e // NW
    ch = epw // K
    npad = -(-n // (NS * KC)) * (NS * KC)
    rpt = npad // NS
    assert e == NW * ch * K and d % 16 == 0 and ch % 2 == 0

    mesh = plsc.VectorSubcoreMesh(core_axis_name="c", subcore_axis_name="s")

    sc_deg = pl.kernel(
        functools.partial(_sc_deg_body, ch, rpt),
        out_type=jax.ShapeDtypeStruct((NC, npad, 16), jnp.float32),
        mesh=mesh,
        scratch_types=[
            pltpu.VMEM_SHARED((npad, 16), jnp.float32),
            pltpu.VMEM((ch, K), jnp.int32),
            pltpu.VMEM((K, 16), jnp.float32),
            pltpu.VMEM((KC, 16), jnp.float32),
        ],
    )

    sc_agg = pl.kernel(
        functools.partial(_sc_agg_body, d, ch, rpt),
        out_type=jax.ShapeDtypeStruct((NC, npad, d), jnp.float32),
        mesh=mesh,
        scratch_types=[
            pltpu.VMEM_SHARED((npad, d), jnp.float32),
            pltpu.VMEM((BLK, K), jnp.int32),
            pltpu.VMEM((BLK, K), jnp.int32),
            pltpu.VMEM((K, d), jnp.float32),
            pltpu.SemaphoreType.DMA,
        ],
    )

    tc_misc = pl.pallas_call(
        functools.partial(_tc_misc_body, n),
        out_shape=jax.ShapeDtypeStruct((n, 1), jnp.float32),
    )
    tc_h2 = pl.pallas_call(
        _tc_h2_body,
        out_shape=jax.ShapeDtypeStruct((n, d), jnp.float32),
    )
    tc_final = pl.pallas_call(
        functools.partial(_tc_final_body, n, G),
        out_shape=jax.ShapeDtypeStruct((G, cdim), jnp.float32),
    )

    row2d = edge_index[0].reshape(e // K, K)
    col2d = edge_index[1].reshape(e // K, K)

    degp = sc_deg(col2d)
    dinv = tc_misc(degp[0], degp[1])
    h2 = tc_h2(x, W.T, dinv)
    parts = sc_agg(h2, row2d, col2d)
    return tc_final(parts[0], parts[1], h2, dinv,
                    b.reshape(1, d), bn_gamma.reshape(1, d),
                    bn_beta.reshape(1, d), batch.reshape(1, n),
                    fc_W.T, fc_b.reshape(1, cdim))


# speedup vs baseline: 36.1759x; 1.3201x over previous
import functools

import jax
import jax.numpy as jnp
from jax import lax
from jax.experimental import pallas as pl
from jax.experimental.pallas import tpu as pltpu
from jax.experimental.pallas import tpu_sc as plsc

NC, NS = 2, 16
NW = NC * NS
K = 125
KC = 64
G = 64


def _sc_deg_body(ch, rpt, col2d, out_hbm, degS, colv, onesv, zbuf):
    c = lax.axis_index("c")
    s = lax.axis_index("s")
    wid = s * NC + c
    nz = rpt // KC

    lane = lax.iota(jnp.int32, 16)
    e1 = jnp.where(lane == 0, jnp.float32(1), jnp.float32(0))
    z16 = jnp.zeros((16,), jnp.float32)

    def fill(r, _):
        onesv[r, :] = e1
        return ()
    lax.fori_loop(0, K, fill, ())

    def fillz(r, _):
        zbuf[r, :] = z16
        return ()
    lax.fori_loop(0, KC, fillz, ())

    for i in range(nz):
        pltpu.sync_copy(zbuf, degS.at[pl.ds(s * rpt + i * KC, KC)])
    plsc.subcore_barrier()

    pltpu.sync_copy(col2d.at[pl.ds(wid * ch, ch)], colv)

    def body(j, _):
        pltpu.sync_copy(onesv, degS.at[colv.at[j]], add=True)
        return ()
    lax.fori_loop(0, ch, body, ())
    plsc.subcore_barrier()

    for i in range(nz):
        base = s * rpt + i * KC
        pltpu.sync_copy(degS.at[pl.ds(base, KC)], zbuf)
        pltpu.sync_copy(zbuf, out_hbm.at[c, pl.ds(base, KC)])


BLK = 8


def _sc_agg_body(d, ch, rpt, h2_hbm, row2d, col2d, out_hbm,
                 aggS, rowv, colv, buf0, buf1, sem0, sem1):
    c = lax.axis_index("c")
    s = lax.axis_index("s")
    wid = s * NC + c
    nz = rpt // KC
    nblk = ch // BLK

    z16 = jnp.zeros((16,), jnp.float32)
    nv = d // 16

    def zero(i, _):
        buf0[i // nv, pl.ds((i % nv) * 16, 16)] = z16
        return ()
    lax.fori_loop(0, KC * nv, zero, ())

    for i in range(nz):
        pltpu.sync_copy(buf0.at[pl.ds(0, KC)],
                        aggS.at[pl.ds(s * rpt + i * KC, KC)])
    plsc.subcore_barrier()

    bufs = (buf0, buf1)
    sems = (sem0, sem1)

    pltpu.sync_copy(row2d.at[pl.ds(wid * ch, BLK)], rowv.at[pl.ds(0, BLK)])
    pltpu.sync_copy(col2d.at[pl.ds(wid * ch, BLK)], colv.at[pl.ds(0, BLK)])
    pltpu.async_copy(h2_hbm.at[rowv.at[0]], buf0, sem0)

    def blk_body(blk, _):
        nb = blk + 1
        half = (blk % 2) * BLK
        nhalf = (nb % 2) * BLK

        @pl.when(nb < nblk)
        def _():
            base = wid * ch + nb * BLK
            pltpu.sync_copy(row2d.at[pl.ds(base, BLK)],
                            rowv.at[pl.ds(nhalf, BLK)])
            pltpu.sync_copy(col2d.at[pl.ds(base, BLK)],
                            colv.at[pl.ds(nhalf, BLK)])

        for t in range(BLK):
            cur, nxt = bufs[t % 2], bufs[(t + 1) % 2]
            csem, nsem = sems[t % 2], sems[(t + 1) % 2]
            if t < BLK - 1:
                pltpu.async_copy(h2_hbm.at[rowv.at[half + t + 1]], nxt, nsem)
            else:
                @pl.when(nb < nblk)
                def _():
                    pltpu.async_copy(h2_hbm.at[rowv.at[nhalf]], nxt, nsem)
            pltpu.make_async_copy(h2_hbm.at[rowv.at[half + t]],
                                  cur, csem).wait()
            pltpu.sync_copy(cur, aggS.at[colv.at[half + t]], add=True)
        return ()
    lax.fori_loop(0, nblk, blk_body, ())
    plsc.subcore_barrier()

    for i in range(nz):
        base = s * rpt + i * KC
        pltpu.sync_copy(aggS.at[pl.ds(base, KC)], buf0.at[pl.ds(0, KC)])
        pltpu.sync_copy(buf0.at[pl.ds(0, KC)], out_hbm.at[c, pl.ds(base, KC)])


def _tc_misc_body(n, d0, d1, o):
    t = d0[...][:n] + d1[...][:n]
    deg = jnp.sum(t, axis=1, keepdims=True) + 1.0
    o[...] = lax.rsqrt(deg)


def _tc_h2_body(x, wt, dinv, o):
    h = jnp.dot(x[...], wt[...], preferred_element_type=jnp.float32)
    o[...] = h * dinv[...]


def _tc_final_body(n, g, p0, p1, h2, dinv, b, gam, bet, batr, fcwt, fcb, o):
    agg = (p0[...][:n] + p1[...][:n] + h2[...]) * dinv[...] + b[...]
    mean = jnp.mean(agg, axis=0, keepdims=True)
    var = jnp.mean((agg - mean) ** 2, axis=0, keepdims=True)
    hn = (agg - mean) * lax.rsqrt(var + 1e-5) * gam[...] + bet[...]
    hr = jnp.maximum(hn, 0.0)
    gid = lax.broadcasted_iota(jnp.int32, (g, 1), 0)
    oh = (gid == batr[...]).astype(jnp.float32)
    sums = jnp.dot(oh, hr, preferred_element_type=jnp.float32)
    cnt = jnp.sum(oh, axis=1, keepdims=True)
    pooled = sums / jnp.maximum(cnt, 1.0)
    logits = jnp.dot(pooled, fcwt[...],
                     preferred_element_type=jnp.float32) + fcb[...]
    m = jnp.max(logits, axis=1, keepdims=True)
    ls = jnp.log(jnp.sum(jnp.exp(logits - m), axis=1, keepdims=True)) + m
    o[...] = logits - ls


def kernel(x, edge_index, batch, W, b, bn_gamma, bn_beta, fc_W, fc_b):
    n, d = x.shape
    e = edge_index.shape[1]
    cdim = fc_W.shape[0]
    epw = e // NW
    ch = epw // K
    npad = -(-n // (NS * KC)) * (NS * KC)
    rpt = npad // NS
    assert e == NW * ch * K and d % 16 == 0 and ch % 2 == 0

    mesh = plsc.VectorSubcoreMesh(core_axis_name="c", subcore_axis_name="s")

    sc_deg = pl.kernel(
        functools.partial(_sc_deg_body, ch, rpt),
        out_type=jax.ShapeDtypeStruct((NC, npad, 16), jnp.float32),
        mesh=mesh,
        scratch_types=[
            pltpu.VMEM_SHARED((npad, 16), jnp.float32),
            pltpu.VMEM((ch, K), jnp.int32),
            pltpu.VMEM((K, 16), jnp.float32),
            pltpu.VMEM((KC, 16), jnp.float32),
        ],
    )

    sc_agg = pl.kernel(
        functools.partial(_sc_agg_body, d, ch, rpt),
        out_type=jax.ShapeDtypeStruct((NC, npad, d), jnp.float32),
        mesh=mesh,
        scratch_types=[
            pltpu.VMEM_SHARED((npad, d), jnp.float32),
            pltpu.VMEM((2 * BLK, K), jnp.int32),
            pltpu.VMEM((2 * BLK, K), jnp.int32),
            pltpu.VMEM((K, d), jnp.float32),
            pltpu.VMEM((K, d), jnp.float32),
            pltpu.SemaphoreType.DMA,
            pltpu.SemaphoreType.DMA,
        ],
    )

    tc_misc = pl.pallas_call(
        functools.partial(_tc_misc_body, n),
        out_shape=jax.ShapeDtypeStruct((n, 1), jnp.float32),
    )
    tc_h2 = pl.pallas_call(
        _tc_h2_body,
        out_shape=jax.ShapeDtypeStruct((n, d), jnp.float32),
    )
    tc_final = pl.pallas_call(
        functools.partial(_tc_final_body, n, G),
        out_shape=jax.ShapeDtypeStruct((G, cdim), jnp.float32),
    )

    row2d = edge_index[0].reshape(e // K, K)
    col2d = edge_index[1].reshape(e // K, K)

    degp = sc_deg(col2d)
    dinv = tc_misc(degp[0], degp[1])
    h2 = tc_h2(x, W.T, dinv)
    parts = sc_agg(h2, row2d, col2d)
    return tc_final(parts[0], parts[1], h2, dinv,
                    b.reshape(1, d), bn_gamma.reshape(1, d),
                    bn_beta.reshape(1, d), batch.reshape(1, n),
                    fc_W.T, fc_b.reshape(1, cdim))
